# baseline (speedup 1.0000x reference)
import functools

import jax
import jax.numpy as jnp
from jax import lax
from jax.experimental import pallas as pl
from jax.experimental.pallas import tpu as pltpu
from jax.experimental.pallas import tpu_sc as plsc

_NUM_SC_CORES = 2
_NUM_SUBCORES = 16
_NW = _NUM_SC_CORES * _NUM_SUBCORES
_IDX_CHUNK = 128


def _make_sc_gather(n_pos, dtype):
    bpw = n_pos // _NW
    nck = bpw // _IDX_CHUNK
    fo = jax.ShapeDtypeStruct((n_pos, 128), dtype)
    mesh = plsc.VectorSubcoreMesh(core_axis_name="c", subcore_axis_name="s")

    @functools.partial(
        pl.kernel,
        mesh=mesh,
        out_type=(fo, fo, fo),
        scratch_types=(
            pltpu.VMEM((nck, _IDX_CHUNK), jnp.int32),
            pltpu.VMEM((nck, _IDX_CHUNK), jnp.int32),
            pltpu.VMEM((nck, _IDX_CHUNK), jnp.int32),
            pltpu.VMEM((bpw, 128), dtype),
            pltpu.VMEM((bpw, 128), dtype),
            pltpu.VMEM((bpw, 128), dtype),
            pltpu.SemaphoreType.DMA,
        ),
    )
    def sc_gather(table_hbm, t_hbm, h_hbm, w_hbm, ot_hbm, oh_hbm, ow_hbm,
                  it_v, ih_v, iw_v, rt_v, rh_v, rw_v, sem):
        wid = lax.axis_index("s") * _NUM_SC_CORES + lax.axis_index("c")
        base = wid * bpw
        row0 = wid * nck
        pltpu.sync_copy(t_hbm.at[pl.ds(row0, nck)], it_v)
        pltpu.sync_copy(h_hbm.at[pl.ds(row0, nck)], ih_v)
        pltpu.sync_copy(w_hbm.at[pl.ds(row0, nck)], iw_v)
        copies = []
        for c in range(nck):
            dst = pl.ds(c * _IDX_CHUNK, _IDX_CHUNK)
            copies.append(pltpu.async_copy(table_hbm.at[it_v.at[c]], rt_v.at[dst], sem))
            copies.append(pltpu.async_copy(table_hbm.at[ih_v.at[c]], rh_v.at[dst], sem))
            copies.append(pltpu.async_copy(table_hbm.at[iw_v.at[c]], rw_v.at[dst], sem))
        for cp in copies:
            cp.wait()
        pltpu.sync_copy(rt_v, ot_hbm.at[pl.ds(base, bpw)])
        pltpu.sync_copy(rh_v, oh_hbm.at[pl.ds(base, bpw)])
        pltpu.sync_copy(rw_v, ow_hbm.at[pl.ds(base, bpw)])

    return sc_gather


def _apply_body(gt_ref, gh_ref, gw_ref, x_ref, o_ref):
    gt = gt_ref[...]
    gh = gh_ref[...]
    gw = gw_ref[...]
    cos_h = jnp.concatenate([gt[:, 0:16], gh[:, 32:56], gw[:, 80:104]], axis=-1)
    sin_h = jnp.concatenate([gt[:, 16:32], gh[:, 56:80], gw[:, 104:128]], axis=-1)
    cos = jnp.concatenate([cos_h, cos_h], axis=-1)[:, None, :]
    sins = jnp.concatenate([-sin_h, sin_h], axis=-1)[:, None, :]
    x = x_ref[...]
    half = x.shape[-1] // 2
    xr = pltpu.roll(x, half, axis=2)
    o_ref[...] = x * cos + xr * sins


def _apply_body_alias(gt_ref, gh_ref, gw_ref, x_ref, prev_ref, o_ref):
    del prev_ref
    _apply_body(gt_ref, gh_ref, gw_ref, x_ref, o_ref)


def kernel(x, input_pos, time_cache, height_cache, width_cache):
    B, S, H, D = x.shape
    n = B * S
    rows = height_cache.shape[0]
    tc = time_cache[:rows]
    table = jnp.concatenate(
        [
            tc[:, 0:16], tc[:, 128:144],
            height_cache[:, 16:40], height_cache[:, 144:168],
            width_cache[:, 40:64], width_cache[:, 168:192],
        ],
        axis=1,
    )
    half = n // 2
    ids = input_pos.reshape(3, 2, half // _IDX_CHUNK, _IDX_CHUNK)
    sc = _make_sc_gather(half, x.dtype)
    gt0, gh0, gw0 = sc(table, ids[0, 0], ids[1, 0], ids[2, 0])
    gt1, gh1, gw1 = sc(table, ids[0, 1], ids[1, 1], ids[2, 1])

    xf = x.reshape(n, H, D)
    lblk = 512
    nblk = half // lblk
    gspec = pl.BlockSpec((lblk, 128), lambda i: (i, 0))
    out0 = pl.pallas_call(
        _apply_body,
        grid=(nblk,),
        in_specs=[
            gspec, gspec, gspec,
            pl.BlockSpec((lblk, H, D), lambda i: (i, 0, 0)),
        ],
        out_specs=pl.BlockSpec((lblk, H, D), lambda i: (i, 0, 0)),
        out_shape=jax.ShapeDtypeStruct((n, H, D), x.dtype),
    )(gt0, gh0, gw0, xf)
    out = pl.pallas_call(
        _apply_body_alias,
        grid=(nblk,),
        in_specs=[
            gspec, gspec, gspec,
            pl.BlockSpec((lblk, H, D), lambda i: (i + nblk, 0, 0)),
            pl.BlockSpec(memory_space=pltpu.MemorySpace.HBM),
        ],
        out_specs=pl.BlockSpec((lblk, H, D), lambda i: (i + nblk, 0, 0)),
        out_shape=jax.ShapeDtypeStruct((n, H, D), x.dtype),
        input_output_aliases={4: 0},
    )(gt1, gh1, gw1, xf, out0)
    return out.reshape(B, S, H, D)


# speedup vs baseline: 1.0131x; 1.0131x over previous
import functools

import jax
import jax.numpy as jnp
from jax import lax
from jax.experimental import pallas as pl
from jax.experimental.pallas import tpu as pltpu
from jax.experimental.pallas import tpu_sc as plsc

_NUM_SC_CORES = 2
_NUM_SUBCORES = 16
_NW = _NUM_SC_CORES * _NUM_SUBCORES
_IDX_CHUNK = 128


def _make_sc_gather(n_pos, dtype):
    bpw = n_pos // _NW
    nck = bpw // _IDX_CHUNK
    fo = jax.ShapeDtypeStruct((n_pos, 128), dtype)
    mesh = plsc.VectorSubcoreMesh(core_axis_name="c", subcore_axis_name="s")

    @functools.partial(
        pl.kernel,
        mesh=mesh,
        out_type=(fo, fo, fo),
        scratch_types=(
            pltpu.VMEM((nck, _IDX_CHUNK), jnp.int32),
            pltpu.VMEM((nck, _IDX_CHUNK), jnp.int32),
            pltpu.VMEM((nck, _IDX_CHUNK), jnp.int32),
            pltpu.VMEM((bpw, 128), dtype),
            pltpu.VMEM((bpw, 128), dtype),
            pltpu.VMEM((bpw, 128), dtype),
            pltpu.SemaphoreType.DMA,
        ),
    )
    def sc_gather(table_hbm, t_hbm, h_hbm, w_hbm, ot_hbm, oh_hbm, ow_hbm,
                  it_v, ih_v, iw_v, rt_v, rh_v, rw_v, sem):
        wid = lax.axis_index("s") * _NUM_SC_CORES + lax.axis_index("c")
        base = wid * bpw
        row0 = wid * nck
        pltpu.sync_copy(t_hbm.at[pl.ds(row0, nck)], it_v)
        pltpu.sync_copy(h_hbm.at[pl.ds(row0, nck)], ih_v)
        pltpu.sync_copy(w_hbm.at[pl.ds(row0, nck)], iw_v)
        copies = []
        for c in range(nck):
            dst = pl.ds(c * _IDX_CHUNK, _IDX_CHUNK)
            copies.append(pltpu.async_copy(table_hbm.at[it_v.at[c]], rt_v.at[dst], sem))
            copies.append(pltpu.async_copy(table_hbm.at[ih_v.at[c]], rh_v.at[dst], sem))
            copies.append(pltpu.async_copy(table_hbm.at[iw_v.at[c]], rw_v.at[dst], sem))
        for cp in copies:
            cp.wait()
        pltpu.sync_copy(rt_v, ot_hbm.at[pl.ds(base, bpw)])
        pltpu.sync_copy(rh_v, oh_hbm.at[pl.ds(base, bpw)])
        pltpu.sync_copy(rw_v, ow_hbm.at[pl.ds(base, bpw)])

    return sc_gather


def _apply_body(gt_ref, gh_ref, gw_ref, x_ref, o_ref):
    gt = gt_ref[...]
    gh = gh_ref[...]
    gw = gw_ref[...]
    cos_h = jnp.concatenate([gt[:, 0:16], gh[:, 32:56], gw[:, 80:104]], axis=-1)
    sin_h = jnp.concatenate([gt[:, 16:32], gh[:, 56:80], gw[:, 104:128]], axis=-1)
    cos = jnp.concatenate([cos_h, cos_h], axis=-1)[:, None, :]
    sins = jnp.concatenate([-sin_h, sin_h], axis=-1)[:, None, :]
    x = x_ref[...]
    half = x.shape[-1] // 2
    xr = pltpu.roll(x, half, axis=2)
    o_ref[...] = x * cos + xr * sins


def _apply_body_alias(gt_ref, gh_ref, gw_ref, x_ref, prev_ref, o_ref):
    del prev_ref
    _apply_body(gt_ref, gh_ref, gw_ref, x_ref, o_ref)


def kernel(x, input_pos, time_cache, height_cache, width_cache):
    B, S, H, D = x.shape
    n = B * S
    rows = height_cache.shape[0]
    tc = time_cache[:rows]
    table = jnp.concatenate(
        [
            tc[:, 0:16], tc[:, 128:144],
            height_cache[:, 16:40], height_cache[:, 144:168],
            width_cache[:, 40:64], width_cache[:, 168:192],
        ],
        axis=1,
    )
    ids = input_pos.reshape(3, n // _IDX_CHUNK, _IDX_CHUNK)
    gt, gh, gw = _make_sc_gather(n, x.dtype)(table, ids[0], ids[1], ids[2])

    xf = x.reshape(n, H, D)
    lblk = 512
    gspec = pl.BlockSpec((lblk, 128), lambda i: (i, 0))
    out = pl.pallas_call(
        _apply_body,
        grid=(n // lblk,),
        in_specs=[
            gspec, gspec, gspec,
            pl.BlockSpec((lblk, H, D), lambda i: (i, 0, 0)),
        ],
        out_specs=pl.BlockSpec((lblk, H, D), lambda i: (i, 0, 0)),
        out_shape=jax.ShapeDtypeStruct((n, H, D), x.dtype),
        compiler_params=pltpu.CompilerParams(
            dimension_semantics=("parallel",),
        ),
    )(gt, gh, gw, xf)
    return out.reshape(B, S, H, D)
